# Initial kernel scaffold; baseline (speedup 1.0000x reference)
import jax
import jax.numpy as jnp
from jax.experimental import pallas as pl


def kernel(x, edge_index, edge_attr, batch, c1_nn_w, c1_nn_b, c1_root, c1_bias, c2_nn_w, c2_nn_b, c2_root, c2_bias, c3_nn_w, c3_nn_b, c3_root, c3_bias, lin1_w, lin1_b, lin2_w, lin2_b):
    raise NotImplementedError("write your pallas kernel here")


# speedup vs baseline: 7.3481x; 7.3481x over previous
import functools

import jax
import jax.numpy as jnp
from jax import lax
from jax.experimental import pallas as pl
from jax.experimental.pallas import tpu as pltpu
from jax.experimental.pallas import tpu_sc as plsc

N = 10000
E = 160000
F_IN = 128
H = 16
G = 64

NC = 2
NS = 16
NW = NC * NS

NPAD = 10240
EPAD = 163840
BLK = 128
NBLK = EPAD // (NW * BLK)
ROWS_W = NPAD // NW
ROWS_S = NPAD // NS
SEG = 72

_f32 = jnp.float32


def _mesh():
    return plsc.VectorSubcoreMesh(
        core_axis_name="c", subcore_axis_name="s",
        num_cores=NC, num_subcores=NS)


@functools.cache
def _build_edge_pass():
  @functools.partial(
    pl.kernel,
    out_type=jax.ShapeDtypeStruct((NC, NPAD, H), _f32),
    mesh=_mesh(),
    scratch_types=[
        pltpu.VMEM((NBLK, BLK), jnp.int32),
        pltpu.VMEM((NBLK, BLK), jnp.int32),
        pltpu.VMEM((NBLK, BLK), _f32),
        pltpu.VMEM((2, BLK, 2 * H), _f32),
        pltpu.VMEM((BLK, H), _f32),
        pltpu.VMEM((ROWS_S, H), _f32),
        pltpu.VMEM_SHARED((NPAD, H), _f32),
        pltpu.SemaphoreType.DMA,
        pltpu.SemaphoreType.DMA,
    ],
    compiler_params=pltpu.CompilerParams(use_tc_tiling_on_sc=False),
)
  def _edge_pass(uv_hbm, src_hbm, dst_hbm, ea_hbm, out_hbm,
                 src_v, dst_v, ea_v, rows, msg, zbuf, acc, sem0, sem1):
    c = lax.axis_index("c")
    s = lax.axis_index("s")
    wid = c * NS + s
    sems = (sem0, sem1)

    @pl.loop(0, ROWS_S)
    def _zero(i):
        zbuf[i] = jnp.zeros((H,), _f32)

    pltpu.sync_copy(zbuf, acc.at[pl.ds(s * ROWS_S, ROWS_S)])
    plsc.subcore_barrier()

    row0 = wid * NBLK
    pltpu.sync_copy(src_hbm.at[pl.ds(row0, NBLK)], src_v)
    pltpu.sync_copy(dst_hbm.at[pl.ds(row0, NBLK)], dst_v)
    pltpu.sync_copy(ea_hbm.at[pl.ds(row0, NBLK)], ea_v)

    pltpu.async_copy(uv_hbm.at[src_v.at[0]], rows.at[0], sem0)
    pltpu.async_copy(uv_hbm.at[src_v.at[1]], rows.at[1], sem1)

    @pl.loop(0, NBLK, step=2)
    def _blocks(b):
        for par in range(2):
            blk = b + par
            pltpu.make_async_copy(
                uv_hbm.at[src_v.at[blk]], rows.at[par], sems[par]).wait()

            @pl.loop(0, BLK // 16)
            def _edges(j):
                evec = ea_v[blk, pl.ds(j * 16, 16)]
                for k in range(16):
                    i = j * 16 + k
                    u = rows[par, i, pl.ds(0, H)]
                    v = rows[par, i, pl.ds(H, H)]
                    msg[i] = evec[k] * u + v

            pltpu.sync_copy(msg, acc.at[dst_v.at[blk]], add=True)

            @pl.when(blk + 2 < NBLK)
            def _next():
                pltpu.async_copy(
                    uv_hbm.at[src_v.at[blk + 2]], rows.at[par], sems[par])

    plsc.subcore_barrier()
    pltpu.sync_copy(acc.at[pl.ds(s * ROWS_S, ROWS_S)],
                    out_hbm.at[c, pl.ds(s * ROWS_S, ROWS_S)])

  return _edge_pass


@functools.cache
def _build_pool():
  @functools.partial(
    pl.kernel,
    out_type=[jax.ShapeDtypeStruct((NW, SEG, H), _f32),
              jax.ShapeDtypeStruct((NW, SEG, H), _f32),
              jax.ShapeDtypeStruct((NW, SEG, H), _f32)],
    mesh=_mesh(),
    scratch_types=[
        pltpu.VMEM((ROWS_W, H), _f32),
        pltpu.VMEM((ROWS_W, H), _f32),
        pltpu.VMEM((ROWS_W, H), _f32),
        pltpu.VMEM((ROWS_W,), jnp.int32),
        pltpu.VMEM((H,), _f32),
        pltpu.VMEM((SEG, H), _f32),
        pltpu.VMEM((SEG, H), _f32),
        pltpu.VMEM((SEG, H), _f32),
    ],
    compiler_params=pltpu.CompilerParams(use_tc_tiling_on_sc=False),
  )
  def _pool(p_hbm, r_hbm, bias_hbm, batch_hbm, sum_hbm, max_hbm, cnt_hbm,
            p0v, p1v, rv, bv, biasv, accs, accm, accc):
    c = lax.axis_index("c")
    s = lax.axis_index("s")
    wid = c * NS + s
    base = wid * ROWS_W

    pltpu.sync_copy(p_hbm.at[0, pl.ds(base, ROWS_W)], p0v)
    pltpu.sync_copy(p_hbm.at[1, pl.ds(base, ROWS_W)], p1v)
    pltpu.sync_copy(r_hbm.at[pl.ds(base, ROWS_W)], rv)
    pltpu.sync_copy(batch_hbm.at[pl.ds(base, ROWS_W)], bv)
    pltpu.sync_copy(bias_hbm, biasv)

    @pl.loop(0, SEG)
    def _init(j):
        accs[j] = jnp.zeros((H,), _f32)
        accm[j] = jnp.full((H,), -1e30, _f32)
        accc[j] = jnp.zeros((H,), _f32)

    @pl.loop(0, ROWS_W // 16)
    def _rows(j):
        bvec = bv[pl.ds(j * 16, 16)]
        for k in range(16):
            i = j * 16 + k
            g = bvec[k]
            h = jnp.maximum(p0v[i] + p1v[i] + rv[i] + biasv[...], 0.0)
            accs[g] = accs[g] + h
            accm[g] = jnp.maximum(accm[g], h)
            accc[g] = accc[g] + jnp.ones((H,), _f32)

    pltpu.sync_copy(accs, sum_hbm.at[wid])
    pltpu.sync_copy(accm, max_hbm.at[wid])
    pltpu.sync_copy(accc, cnt_hbm.at[wid])

  return _pool


def _tc1_body(x_ref, wuv_ref, wr_ref, uv_ref, r_ref):
    x = x_ref[...]
    uv_ref[...] = jnp.dot(x, wuv_ref[...], preferred_element_type=_f32)
    r_ref[pl.ds(0, N), :] = jnp.dot(x, wr_ref[...], preferred_element_type=_f32)
    r_ref[pl.ds(N, NPAD - N), :] = jnp.zeros((NPAD - N, H), _f32)


def _tc_combine_body(p_ref, r_ref, bias_ref, wuv_ref, wr_ref, uv_ref, rn_ref):
    h = jax.nn.relu(p_ref[0, pl.ds(0, N), :] + p_ref[1, pl.ds(0, N), :]
                    + r_ref[pl.ds(0, N), :] + bias_ref[...])
    uv_ref[...] = jnp.dot(h, wuv_ref[...], preferred_element_type=_f32)
    rn_ref[pl.ds(0, N), :] = jnp.dot(h, wr_ref[...], preferred_element_type=_f32)
    rn_ref[pl.ds(N, NPAD - N), :] = jnp.zeros((NPAD - N, H), _f32)


def _tc_final_body(s_ref, m_ref, c_ref, l1w_ref, l1b_ref, l2w_ref, l2b_ref,
                   out_ref):
    seg_s = jnp.sum(s_ref[...], axis=0)[:G, :]
    seg_m = jnp.max(m_ref[...], axis=0)[:G, :]
    cnt = jnp.sum(c_ref[...], axis=0)[:G, :1]
    mean = seg_s / jnp.maximum(cnt, 1.0)
    seg_m = jnp.where(cnt > 0, seg_m, 0.0)
    z = jnp.concatenate([seg_s, mean, seg_m], axis=1)
    z = jax.nn.relu(jnp.dot(z, l1w_ref[...], preferred_element_type=_f32)
                    + l1b_ref[...])
    z = jnp.dot(z, l2w_ref[...], preferred_element_type=_f32) + l2b_ref[...]
    zm = z - jnp.max(z, axis=1, keepdims=True)
    out_ref[...] = zm - jnp.log(jnp.sum(jnp.exp(zm), axis=1, keepdims=True))


_tc1 = pl.pallas_call(
    _tc1_body,
    out_shape=[jax.ShapeDtypeStruct((N, 2 * H), _f32),
               jax.ShapeDtypeStruct((NPAD, H), _f32)])

_tc_combine = pl.pallas_call(
    _tc_combine_body,
    out_shape=[jax.ShapeDtypeStruct((N, 2 * H), _f32),
               jax.ShapeDtypeStruct((NPAD, H), _f32)])

_tc_final = pl.pallas_call(
    _tc_final_body,
    out_shape=jax.ShapeDtypeStruct((G, 2), _f32))


def kernel(x, edge_index, edge_attr, batch,
           c1_nn_w, c1_nn_b, c1_root, c1_bias,
           c2_nn_w, c2_nn_b, c2_root, c2_bias,
           c3_nn_w, c3_nn_b, c3_root, c3_bias,
           lin1_w, lin1_b, lin2_w, lin2_b):
    src = edge_index[0]
    dst = edge_index[1]
    ea = edge_attr[:, 0]
    epad = EPAD - E
    src_p = jnp.concatenate(
        [src, jnp.zeros((epad,), jnp.int32)]).reshape(NW * NBLK, BLK)
    dst_p = jnp.concatenate(
        [dst, jnp.full((epad,), N, jnp.int32)]).reshape(NW * NBLK, BLK)
    ea_p = jnp.concatenate(
        [ea, jnp.zeros((epad,), _f32)]).reshape(NW * NBLK, BLK)
    batch_p = jnp.concatenate([batch, jnp.full((NPAD - N,), G, jnp.int32)])

    wuv1 = jnp.concatenate(
        [c1_nn_w.reshape(F_IN, H), c1_nn_b.reshape(F_IN, H)], axis=1)
    wuv2 = jnp.concatenate(
        [c2_nn_w.reshape(H, H), c2_nn_b.reshape(H, H)], axis=1)
    wuv3 = jnp.concatenate(
        [c3_nn_w.reshape(H, H), c3_nn_b.reshape(H, H)], axis=1)

    uv1, r1 = _tc1(x, wuv1, c1_root)
    edge_pass = _build_edge_pass()
    p1 = edge_pass(uv1, src_p, dst_p, ea_p)
    uv2, r2 = _tc_combine(p1, r1, c1_bias.reshape(1, H), wuv2, c2_root)
    p2 = edge_pass(uv2, src_p, dst_p, ea_p)
    uv3, r3 = _tc_combine(p2, r2, c2_bias.reshape(1, H), wuv3, c3_root)
    p3 = edge_pass(uv3, src_p, dst_p, ea_p)
    seg_s, seg_m, seg_c = _build_pool()(p3, r3, c3_bias, batch_p)
    return _tc_final(seg_s, seg_m, seg_c, lin1_w, lin1_b.reshape(1, H),
                     lin2_w, lin2_b.reshape(1, 2))
